# baseline (speedup 1.0000x reference)
import jax
import jax.numpy as jnp
from jax.experimental import pallas as pl
from jax.experimental.pallas import tpu as pltpu

_N = 10000
_F = 128
_BM = 200


def _gcn_kernel(x_ref, w1_ref, b1_ref, w2_ref, b2_ref, g_ref,
                out_ref, s1_scr, s2_scr):
    p = pl.program_id(0)
    i = pl.program_id(1)

    @pl.when((p == 0) & (i == 0))
    def _():
        s1_scr[...] = jnp.dot(
            x_ref[...], w1_ref[...],
            preferred_element_type=jnp.float32,
            precision=jax.lax.Precision.HIGHEST,
        ).astype(jnp.bfloat16)

    gb = g_ref[...].astype(jnp.bfloat16)

    @pl.when(p == 0)
    def _():
        acc = jnp.dot(gb, s1_scr[...], preferred_element_type=jnp.float32)
        h = jnp.maximum(acc + b1_ref[...], 0.0)
        s2_scr[pl.ds(i * _BM, _BM), :] = jnp.dot(
            h, w2_ref[...],
            preferred_element_type=jnp.float32,
            precision=jax.lax.Precision.HIGHEST,
        ).astype(jnp.bfloat16)

    @pl.when(p == 1)
    def _():
        out_ref[...] = jnp.dot(
            gb, s2_scr[...], preferred_element_type=jnp.float32
        ) + b2_ref[...]


def kernel(g, x, W1, b1, W2, b2):
    n_blocks = _N // _BM
    return pl.pallas_call(
        _gcn_kernel,
        grid=(2, n_blocks),
        in_specs=[
            pl.BlockSpec((_N, _F), lambda p, i: (0, 0)),
            pl.BlockSpec((_F, _F), lambda p, i: (0, 0)),
            pl.BlockSpec((1, _F), lambda p, i: (0, 0)),
            pl.BlockSpec((_F, _F), lambda p, i: (0, 0)),
            pl.BlockSpec((1, _F), lambda p, i: (0, 0)),
            pl.BlockSpec((_BM, _N), lambda p, i: (i, 0)),
        ],
        out_specs=pl.BlockSpec((_BM, _F), lambda p, i: (i * p, 0)),
        out_shape=jax.ShapeDtypeStruct((_N, _F), jnp.float32),
        scratch_shapes=[
            pltpu.VMEM((_N, _F), jnp.bfloat16),
            pltpu.VMEM((_N, _F), jnp.bfloat16),
        ],
        compiler_params=pltpu.CompilerParams(
            dimension_semantics=("arbitrary", "arbitrary"),
        ),
    )(x, W1, b1.reshape(1, _F), W2, b2.reshape(1, _F), g)


# speedup vs baseline: 1.0408x; 1.0408x over previous
import jax
import jax.numpy as jnp
from jax.experimental import pallas as pl
from jax.experimental.pallas import tpu as pltpu

_N = 10000
_F = 128
_BM = 200


def _s1_kernel(x_ref, w1_ref, s1_ref):
    s1_ref[...] = jnp.dot(
        x_ref[...], w1_ref[...],
        preferred_element_type=jnp.float32,
        precision=jax.lax.Precision.HIGHEST,
    )


def _layer1_kernel(s1_ref, b1_ref, w2_ref, g_ref, s2_ref):
    acc = jnp.dot(g_ref[...], s1_ref[...], preferred_element_type=jnp.float32)
    h = jnp.maximum(acc + b1_ref[...], 0.0)
    s2_ref[...] = jnp.dot(
        h, w2_ref[...],
        preferred_element_type=jnp.float32,
        precision=jax.lax.Precision.HIGHEST,
    )


def _layer2_kernel(s2_ref, b2_ref, g_ref, out_ref):
    acc = jnp.dot(g_ref[...], s2_ref[...], preferred_element_type=jnp.float32)
    out_ref[...] = acc + b2_ref[...]


def kernel(g, x, W1, b1, W2, b2):
    n_blocks = _N // _BM

    s1 = pl.pallas_call(
        _s1_kernel,
        out_shape=jax.ShapeDtypeStruct((_N, _F), jnp.float32),
    )(x, W1)

    s2 = pl.pallas_call(
        _layer1_kernel,
        grid=(n_blocks,),
        in_specs=[
            pl.BlockSpec((_N, _F), lambda i: (0, 0)),
            pl.BlockSpec((1, _F), lambda i: (0, 0)),
            pl.BlockSpec((_F, _F), lambda i: (0, 0)),
            pl.BlockSpec((_BM, _N), lambda i: (i, 0)),
        ],
        out_specs=pl.BlockSpec((_BM, _F), lambda i: (i, 0)),
        out_shape=jax.ShapeDtypeStruct((_N, _F), jnp.float32),
        compiler_params=pltpu.CompilerParams(
            dimension_semantics=("parallel",),
        ),
    )(s1, b1.reshape(1, _F), W2, g)

    out = pl.pallas_call(
        _layer2_kernel,
        grid=(n_blocks,),
        in_specs=[
            pl.BlockSpec((_N, _F), lambda i: (0, 0)),
            pl.BlockSpec((1, _F), lambda i: (0, 0)),
            pl.BlockSpec((_BM, _N), lambda i: (i, 0)),
        ],
        out_specs=pl.BlockSpec((_BM, _F), lambda i: (i, 0)),
        out_shape=jax.ShapeDtypeStruct((_N, _F), jnp.float32),
        compiler_params=pltpu.CompilerParams(
            dimension_semantics=("parallel",),
        ),
    )(s2, b2.reshape(1, _F), g)

    return out


# speedup vs baseline: 1.1094x; 1.0660x over previous
import jax
import jax.numpy as jnp
from jax.experimental import pallas as pl
from jax.experimental.pallas import tpu as pltpu

_N = 10000
_F = 128
_BM = 200


def _layer1_kernel(x_ref, w1_ref, b1_ref, w2_ref, g_ref, s2_ref, s1_scr):
    @pl.when(pl.program_id(0) == 0)
    def _():
        s1_scr[...] = jnp.dot(
            x_ref[...], w1_ref[...], preferred_element_type=jnp.float32
        )

    acc = jnp.dot(g_ref[...], s1_scr[...], preferred_element_type=jnp.float32)
    h = jnp.maximum(acc + b1_ref[...], 0.0)
    s2_ref[...] = jnp.dot(h, w2_ref[...], preferred_element_type=jnp.float32)


def _layer2_kernel(s2_ref, b2_ref, g_ref, out_ref):
    acc = jnp.dot(g_ref[...], s2_ref[...], preferred_element_type=jnp.float32)
    out_ref[...] = acc + b2_ref[...]


def kernel(g, x, W1, b1, W2, b2):
    n_blocks = _N // _BM

    s2 = pl.pallas_call(
        _layer1_kernel,
        grid=(n_blocks,),
        in_specs=[
            pl.BlockSpec((_N, _F), lambda i: (0, 0)),
            pl.BlockSpec((_F, _F), lambda i: (0, 0)),
            pl.BlockSpec((1, _F), lambda i: (0, 0)),
            pl.BlockSpec((_F, _F), lambda i: (0, 0)),
            pl.BlockSpec((_BM, _N), lambda i: (i, 0)),
        ],
        out_specs=pl.BlockSpec((_BM, _F), lambda i: (i, 0)),
        out_shape=jax.ShapeDtypeStruct((_N, _F), jnp.float32),
        scratch_shapes=[pltpu.VMEM((_N, _F), jnp.float32)],
        compiler_params=pltpu.CompilerParams(
            dimension_semantics=("arbitrary",),
        ),
    )(x, W1, b1.reshape(1, _F), W2, g)

    out = pl.pallas_call(
        _layer2_kernel,
        grid=(n_blocks,),
        in_specs=[
            pl.BlockSpec((_N, _F), lambda i: (0, 0)),
            pl.BlockSpec((1, _F), lambda i: (0, 0)),
            pl.BlockSpec((_BM, _N), lambda i: (i, 0)),
        ],
        out_specs=pl.BlockSpec((_BM, _F), lambda i: (i, 0)),
        out_shape=jax.ShapeDtypeStruct((_N, _F), jnp.float32),
        compiler_params=pltpu.CompilerParams(
            dimension_semantics=("parallel",),
        ),
    )(s2, b2.reshape(1, _F), g)

    return out


# speedup vs baseline: 1.1218x; 1.0111x over previous
import jax
import jax.numpy as jnp
from jax.experimental import pallas as pl
from jax.experimental.pallas import tpu as pltpu

_N = 10000
_F = 128
_BM = 400


def _layer1_kernel(x_ref, w1_ref, b1_ref, w2_ref, g_ref, s2_ref, s1_scr):
    @pl.when(pl.program_id(0) == 0)
    def _():
        s1_scr[...] = jnp.dot(
            x_ref[...], w1_ref[...], preferred_element_type=jnp.float32
        )

    acc = jnp.dot(g_ref[...], s1_scr[...], preferred_element_type=jnp.float32)
    h = jnp.maximum(acc + b1_ref[...], 0.0)
    s2_ref[...] = jnp.dot(h, w2_ref[...], preferred_element_type=jnp.float32)


def _layer2_kernel(s2_ref, b2_ref, g_ref, out_ref):
    acc = jnp.dot(g_ref[...], s2_ref[...], preferred_element_type=jnp.float32)
    out_ref[...] = acc + b2_ref[...]


def kernel(g, x, W1, b1, W2, b2):
    n_blocks = _N // _BM

    s2 = pl.pallas_call(
        _layer1_kernel,
        grid=(n_blocks,),
        in_specs=[
            pl.BlockSpec((_N, _F), lambda i: (0, 0)),
            pl.BlockSpec((_F, _F), lambda i: (0, 0)),
            pl.BlockSpec((1, _F), lambda i: (0, 0)),
            pl.BlockSpec((_F, _F), lambda i: (0, 0)),
            pl.BlockSpec((_BM, _N), lambda i: (i, 0)),
        ],
        out_specs=pl.BlockSpec((_BM, _F), lambda i: (i, 0)),
        out_shape=jax.ShapeDtypeStruct((_N, _F), jnp.float32),
        scratch_shapes=[pltpu.VMEM((_N, _F), jnp.float32)],
        compiler_params=pltpu.CompilerParams(
            dimension_semantics=("arbitrary",),
        ),
    )(x, W1, b1.reshape(1, _F), W2, g)

    out = pl.pallas_call(
        _layer2_kernel,
        grid=(n_blocks,),
        in_specs=[
            pl.BlockSpec((_N, _F), lambda i: (0, 0)),
            pl.BlockSpec((1, _F), lambda i: (0, 0)),
            pl.BlockSpec((_BM, _N), lambda i: (i, 0)),
        ],
        out_specs=pl.BlockSpec((_BM, _F), lambda i: (i, 0)),
        out_shape=jax.ShapeDtypeStruct((_N, _F), jnp.float32),
        compiler_params=pltpu.CompilerParams(
            dimension_semantics=("parallel",),
        ),
    )(s2, b2.reshape(1, _F), g)

    return out


# speedup vs baseline: 1.1578x; 1.0321x over previous
import jax
import jax.numpy as jnp
from jax.experimental import pallas as pl
from jax.experimental.pallas import tpu as pltpu

_N = 10000
_F = 128
_BM = 400
_NB = _N // _BM


def _gcn_kernel(x_ref, w1_ref, b1_ref, w2_ref, b2_ref, g_ref,
                out_ref, s1_scr, s2_scr):
    i = pl.program_id(0)

    @pl.when(i == 0)
    def _():
        s1_scr[...] = jnp.dot(
            x_ref[...], w1_ref[...], preferred_element_type=jnp.float32
        )

    @pl.when(i < _NB)
    def _():
        acc = jnp.dot(
            g_ref[...], s1_scr[...], preferred_element_type=jnp.float32
        )
        h = jnp.maximum(acc + b1_ref[...], 0.0)
        s2_scr[pl.ds(i * _BM, _BM), :] = jnp.dot(
            h, w2_ref[...], preferred_element_type=jnp.float32
        )

    @pl.when(i >= _NB)
    def _():
        out_ref[...] = jnp.dot(
            g_ref[...], s2_scr[...], preferred_element_type=jnp.float32
        ) + b2_ref[...]


def kernel(g, x, W1, b1, W2, b2):
    return pl.pallas_call(
        _gcn_kernel,
        grid=(2 * _NB,),
        in_specs=[
            pl.BlockSpec((_N, _F), lambda i: (0, 0)),
            pl.BlockSpec((_F, _F), lambda i: (0, 0)),
            pl.BlockSpec((1, _F), lambda i: (0, 0)),
            pl.BlockSpec((_F, _F), lambda i: (0, 0)),
            pl.BlockSpec((1, _F), lambda i: (0, 0)),
            pl.BlockSpec((_BM, _N), lambda i: (i % _NB, 0)),
        ],
        out_specs=pl.BlockSpec(
            (_BM, _F), lambda i: ((i // _NB) * (i - _NB), 0)
        ),
        out_shape=jax.ShapeDtypeStruct((_N, _F), jnp.float32),
        scratch_shapes=[
            pltpu.VMEM((_N, _F), jnp.float32),
            pltpu.VMEM((_N, _F), jnp.float32),
        ],
        compiler_params=pltpu.CompilerParams(
            dimension_semantics=("arbitrary",),
        ),
    )(x, W1, b1.reshape(1, _F), W2, b2.reshape(1, _F), g)
